# baseline (speedup 1.0000x reference)
import functools

import jax
import jax.numpy as jnp
from jax import lax
from jax.experimental import pallas as pl
from jax.experimental.pallas import tpu as pltpu
from jax.experimental.pallas import tpu_sc as plsc

B = 4096
F = 26
V = 100000
D = 32

NC = 2
NS = 16
NW = NC * NS
ROWS = B * F
PER_W = ROWS // NW
GC = 128
NGC = B // GC
LCHUNK = 128
NLCHUNK = PER_W // LCHUNK


def _sc_emb_body(tok_hbm, tbl_hbm, e_out, idx_a, idx_b, out_a, out_b,
                 sem_a, sem_b):
    wid = lax.axis_index("s") * NC + lax.axis_index("c")

    def issue(f, idx_v, out_v, sem):
        rowid = f * D + wid
        return [
            pltpu.async_copy(
                tbl_hbm.at[rowid].at[idx_v.at[pl.ds(c * GC, GC)]],
                out_v.at[pl.ds(c * GC, GC)], sem)
            for c in range(NGC)
        ]

    pltpu.sync_copy(tok_hbm.at[pl.ds(0, B)], idx_a)
    pending = issue(0, idx_a, out_a, sem_a)

    for f in range(F):
        nxt_pending = None
        nxt = f + 1
        if nxt < F:
            idx_n, out_n, sem_n = (idx_b, out_b, sem_b) if nxt % 2 else (
                idx_a, out_a, sem_a)
            pltpu.sync_copy(tok_hbm.at[pl.ds(nxt * B, B)], idx_n)
            nxt_pending = issue(nxt, idx_n, out_n, sem_n)
        out_c = out_b if f % 2 else out_a
        for cp in pending:
            cp.wait()
        pending = nxt_pending
        pltpu.sync_copy(out_c, e_out.at[pl.ds((f * D + wid) * B, B)])


def _sc_lin_body(idx_hbm, lin_hbm, lin_out, idx_v, lin_v, sem):
    wid = lax.axis_index("s") * NC + lax.axis_index("c")
    base = wid * PER_W
    pltpu.sync_copy(idx_hbm.at[pl.ds(base, PER_W)], idx_v)

    def step(c, carry):
        off = c * LCHUNK
        idx_c = idx_v.at[pl.ds(off, LCHUNK)]
        pltpu.async_copy(lin_hbm.at[idx_c], lin_v, sem).wait()
        pltpu.sync_copy(lin_v, lin_out.at[pl.ds(base + off, LCHUNK)])
        return carry

    lax.fori_loop(0, NLCHUNK, step, 0)


def _tc_fm_body(e_ref, lin_ref, ra_ref, rb_ref, bias_ref, out_ref):
    d = pl.program_id(0)
    r = 0.5 * (ra_ref[...] + rb_ref[...])
    ii = lax.broadcasted_iota(jnp.int32, (F, F), 0)
    jj = lax.broadcasted_iota(jnp.int32, (F, F), 1)
    rs = jnp.where(ii == jj, 0.0, r)
    e = e_ref[...]
    t = jnp.dot(rs, e, preferred_element_type=jnp.float32)
    u = jnp.sum(t * e, axis=0, keepdims=True)

    @pl.when(d == 0)
    def _():
        linear = jnp.sum(lin_ref[...], axis=0, keepdims=True)
        out_ref[...] = u + linear + bias_ref[0, 0]

    @pl.when(d != 0)
    def _():
        out_ref[...] += u


def kernel(token_ids, emb_tables, lin_tables, r_raw, bias):
    tok = token_ids.astype(jnp.int32)
    tokT = tok.T
    tokv = tokT.reshape(ROWS)
    idx1d = ((jnp.arange(F, dtype=jnp.int32) * V)[:, None] + tokT).reshape(ROWS)

    tblT = emb_tables.transpose(0, 2, 1).reshape(F * D, V)
    lin_flat = lin_tables.reshape(F * V)

    mesh = plsc.VectorSubcoreMesh(core_axis_name="c", subcore_axis_name="s")
    gather_e = pl.kernel(
        _sc_emb_body,
        out_type=jax.ShapeDtypeStruct((F * D * B,), jnp.float32),
        mesh=mesh,
        compiler_params=pltpu.CompilerParams(use_tc_tiling_on_sc=False),
        scratch_types=[
            pltpu.VMEM((B,), jnp.int32),
            pltpu.VMEM((B,), jnp.int32),
            pltpu.VMEM((B,), jnp.float32),
            pltpu.VMEM((B,), jnp.float32),
            pltpu.SemaphoreType.DMA,
            pltpu.SemaphoreType.DMA,
        ],
    )
    e_rows = gather_e(tokv, tblT)

    gather_l = pl.kernel(
        _sc_lin_body,
        out_type=jax.ShapeDtypeStruct((ROWS,), jnp.float32),
        mesh=mesh,
        compiler_params=pltpu.CompilerParams(use_tc_tiling_on_sc=False),
        scratch_types=[
            pltpu.VMEM((PER_W,), jnp.int32),
            pltpu.VMEM((LCHUNK,), jnp.float32),
            pltpu.SemaphoreType.DMA,
        ],
    )
    lin_rows = gather_l(idx1d, lin_flat)

    e_mat = e_rows.reshape(F, D * B)
    lin_mat = lin_rows.reshape(F, B)

    out2 = pl.pallas_call(
        _tc_fm_body,
        grid=(D,),
        in_specs=[
            pl.BlockSpec((F, B), lambda i: (0, i)),
            pl.BlockSpec((F, B), lambda i: (0, 0)),
            pl.BlockSpec((F, F), lambda i: (0, 0)),
            pl.BlockSpec((F, F), lambda i: (0, 0)),
            pl.BlockSpec((1, 1), lambda i: (0, 0)),
        ],
        out_specs=pl.BlockSpec((1, B), lambda i: (0, 0)),
        out_shape=jax.ShapeDtypeStruct((1, B), jnp.float32),
    )(e_mat, lin_mat, r_raw, r_raw.T, bias.reshape(1, 1))

    return out2.reshape(B)


# speedup vs baseline: 1.1681x; 1.1681x over previous
import functools

import jax
import jax.numpy as jnp
from jax import lax
from jax.experimental import pallas as pl
from jax.experimental.pallas import tpu as pltpu
from jax.experimental.pallas import tpu_sc as plsc

B = 4096
F = 26
V = 100000
D = 32

NC = 2
NS = 16
NW = NC * NS
ROWS = B * F
PER_W = ROWS // NW
GCHUNK = 32
NPAIR = PER_W // (2 * GCHUNK)
LCHUNK = 128
NLCHUNK = PER_W // LCHUNK

_GRP_BYTES = 8 * D * 4


def _issue(tbl_hbm, idxg_v, grp_v, sem, off):
    for jb in range(GCHUNK // 16):
        gvec = idxg_v[pl.ds(off + jb * 16, 16)]
        for l in range(16):
            pltpu.async_copy(tbl_hbm.at[gvec[l]], grp_v.at[jb * 16 + l], sem)


def _drain(tbl_hbm, grp_v, sem):
    pltpu.make_async_copy(tbl_hbm.at[pl.ds(0, GCHUNK)], grp_v, sem).wait()


def _select_write(s_v, grp_v, sel_v, e_out, lane, lane32, base, off):
    for jb in range(GCHUNK // 16):
        jvec = lane + jb * 16
        svec = s_v[pl.ds(off + jb * 16, 16)]
        for d in range(D):
            dvec = jnp.full((16,), d, jnp.int32)
            v = plsc.load_gather(grp_v, [jvec, svec, dvec])
            plsc.store_scatter(sel_v, [lane32 + (jb * 16 * D + d)], v)
    pltpu.sync_copy(sel_v, e_out.at[pl.ds((base + off) * D, GCHUNK * D)])


def _sc_emb_body(idxg_hbm, idxs_hbm, tbl_hbm, e_out, idxg_v, s_v,
                 grp_a, grp_b, sel_a, sel_b, sem_a, sem_b):
    wid = lax.axis_index("s") * NC + lax.axis_index("c")
    base = wid * PER_W
    pltpu.sync_copy(idxg_hbm.at[pl.ds(base, PER_W)], idxg_v)
    pltpu.sync_copy(idxs_hbm.at[pl.ds(base, PER_W)], s_v)

    lane = lax.broadcasted_iota(jnp.int32, (16,), 0)
    lane32 = lane * D

    _issue(tbl_hbm, idxg_v, grp_a, sem_a, 0)

    def step(i, carry):
        off_a = 2 * i * GCHUNK
        off_b = off_a + GCHUNK
        off_next = off_b + GCHUNK
        _issue(tbl_hbm, idxg_v, grp_b, sem_b, off_b)
        _drain(tbl_hbm, grp_a, sem_a)
        _select_write(s_v, grp_a, sel_a, e_out, lane, lane32, base, off_a)

        @pl.when(off_next < PER_W)
        def _():
            _issue(tbl_hbm, idxg_v, grp_a, sem_a, off_next)

        _drain(tbl_hbm, grp_b, sem_b)
        _select_write(s_v, grp_b, sel_b, e_out, lane, lane32, base, off_b)
        return carry

    lax.fori_loop(0, NPAIR, step, 0)


def _sc_lin_body(idx_hbm, lin_hbm, lin_out, idx_v, lin_v, sem):
    wid = lax.axis_index("s") * NC + lax.axis_index("c")
    base = wid * PER_W
    pltpu.sync_copy(idx_hbm.at[pl.ds(base, PER_W)], idx_v)

    def step(c, carry):
        off = c * LCHUNK
        idx_c = idx_v.at[pl.ds(off, LCHUNK)]
        pltpu.async_copy(lin_hbm.at[idx_c], lin_v, sem).wait()
        pltpu.sync_copy(lin_v, lin_out.at[pl.ds(base + off, LCHUNK)])
        return carry

    lax.fori_loop(0, NLCHUNK, step, 0)


def _tc_fm_body(e_ref, lin_ref, ra_ref, rb_ref, m_ref, bias_ref, out_ref):
    r = 0.5 * (ra_ref[...] + rb_ref[...])
    ii = lax.broadcasted_iota(jnp.int32, (F, F), 0)
    jj = lax.broadcasted_iota(jnp.int32, (F, F), 1)
    rs = jnp.where(ii == jj, 0.0, r)
    e = e_ref[...]
    t = jnp.dot(rs, e, preferred_element_type=jnp.float32)
    u = jnp.sum(t * e, axis=0, keepdims=True)
    inter = jnp.dot(u, m_ref[...], preferred_element_type=jnp.float32)
    linear = jnp.sum(lin_ref[...], axis=0, keepdims=True)
    out_ref[...] = (inter + linear + bias_ref[0, 0]).reshape(1, 1, 128)


def kernel(token_ids, emb_tables, lin_tables, r_raw, bias):
    tok = token_ids.astype(jnp.int32)
    tokT = tok.T
    idx1d = ((jnp.arange(F, dtype=jnp.int32) * V)[:, None] + tokT).reshape(ROWS)
    idx_g = idx1d >> 3
    idx_s = idx1d & 7

    tbl3 = emb_tables.reshape(F * V // 8, 8, D)
    lin_flat = lin_tables.reshape(F * V)

    mesh = plsc.VectorSubcoreMesh(core_axis_name="c", subcore_axis_name="s")
    gather_e = pl.kernel(
        _sc_emb_body,
        out_type=jax.ShapeDtypeStruct((ROWS * D,), jnp.float32),
        mesh=mesh,
        compiler_params=pltpu.CompilerParams(use_tc_tiling_on_sc=True,
                                             needs_layout_passes=False),
        scratch_types=[
            pltpu.VMEM((PER_W,), jnp.int32),
            pltpu.VMEM((PER_W,), jnp.int32),
            pltpu.VMEM((GCHUNK, 8, D), jnp.float32),
            pltpu.VMEM((GCHUNK, 8, D), jnp.float32),
            pltpu.VMEM((GCHUNK * D,), jnp.float32),
            pltpu.VMEM((GCHUNK * D,), jnp.float32),
            pltpu.SemaphoreType.DMA,
            pltpu.SemaphoreType.DMA,
        ],
    )
    e_rows = gather_e(idx_g, idx_s, tbl3)

    gather_l = pl.kernel(
        _sc_lin_body,
        out_type=jax.ShapeDtypeStruct((ROWS,), jnp.float32),
        mesh=mesh,
        compiler_params=pltpu.CompilerParams(use_tc_tiling_on_sc=False),
        scratch_types=[
            pltpu.VMEM((PER_W,), jnp.int32),
            pltpu.VMEM((LCHUNK,), jnp.float32),
            pltpu.SemaphoreType.DMA,
        ],
    )
    lin_rows = gather_l(idx1d, lin_flat)

    e_mat = e_rows.reshape(F, B * D)
    lin_mat = lin_rows.reshape(F, B)

    msel = ((jnp.arange(128 * D, dtype=jnp.int32) // D)[:, None]
            == jnp.arange(128, dtype=jnp.int32)[None, :]).astype(jnp.float32)

    out3 = pl.pallas_call(
        _tc_fm_body,
        grid=(B // 128,),
        in_specs=[
            pl.BlockSpec((F, 128 * D), lambda i: (0, i)),
            pl.BlockSpec((F, 128), lambda i: (0, i)),
            pl.BlockSpec((F, F), lambda i: (0, 0)),
            pl.BlockSpec((F, F), lambda i: (0, 0)),
            pl.BlockSpec((128 * D, 128), lambda i: (0, 0)),
            pl.BlockSpec((1, 1), lambda i: (0, 0)),
        ],
        out_specs=pl.BlockSpec((1, 1, 128), lambda i: (i, 0, 0)),
        out_shape=jax.ShapeDtypeStruct((B // 128, 1, 128), jnp.float32),
    )(e_mat, lin_mat, r_raw, r_raw.T, msel, bias.reshape(1, 1))

    return out3.reshape(B)


# speedup vs baseline: 1.1682x; 1.0001x over previous
import jax
import jax.numpy as jnp
from jax import lax
from jax.experimental import pallas as pl
from jax.experimental.pallas import tpu as pltpu
from jax.experimental.pallas import tpu_sc as plsc

B = 4096
F = 26
V = 100000
D = 32

NC = 2
NS = 16
NW = NC * NS
ROWS = B * F
PER_W = ROWS // NW
GCHUNK = 32
NPAIR = PER_W // (2 * GCHUNK)
LCHUNK = 128
NLCHUNK = PER_W // LCHUNK


def _issue(tbl_hbm, idxg_v, grp_v, sem, off):
    for jb in range(GCHUNK // 16):
        gvec = idxg_v[pl.ds(off + jb * 16, 16)]
        for l in range(16):
            pltpu.async_copy(tbl_hbm.at[gvec[l]], grp_v.at[jb * 16 + l], sem)


def _drain(tbl_hbm, grp_v, sem):
    pltpu.make_async_copy(tbl_hbm.at[pl.ds(0, GCHUNK)], grp_v, sem).wait()


def _select_write(s_v, grp_v, sel_v, e_out, lane, lane32, base, off):
    for jb in range(GCHUNK // 16):
        jvec = lane + jb * 16
        svec = s_v[pl.ds(off + jb * 16, 16)]
        for d in range(D):
            dvec = jnp.full((16,), d, jnp.int32)
            v = plsc.load_gather(grp_v, [jvec, svec, dvec])
            plsc.store_scatter(sel_v, [lane32 + (jb * 16 * D + d)], v)
    pltpu.sync_copy(sel_v, e_out.at[pl.ds((base + off) * D, GCHUNK * D)])


def _sc_emb_body(idxg_hbm, idxs_hbm, tbl_hbm, e_out, idxg_v, s_v,
                 grp_a, grp_b, sel_a, sel_b, sem_a, sem_b):
    wid = lax.axis_index("s") * NC + lax.axis_index("c")
    base = wid * PER_W
    pltpu.sync_copy(idxg_hbm.at[pl.ds(base, PER_W)], idxg_v)
    pltpu.sync_copy(idxs_hbm.at[pl.ds(base, PER_W)], s_v)

    lane = lax.broadcasted_iota(jnp.int32, (16,), 0)
    lane32 = lane * D

    _issue(tbl_hbm, idxg_v, grp_a, sem_a, 0)

    def step(i, carry):
        off_a = 2 * i * GCHUNK
        off_b = off_a + GCHUNK
        off_next = off_b + GCHUNK
        _issue(tbl_hbm, idxg_v, grp_b, sem_b, off_b)
        _drain(tbl_hbm, grp_a, sem_a)
        _select_write(s_v, grp_a, sel_a, e_out, lane, lane32, base, off_a)

        @pl.when(off_next < PER_W)
        def _():
            _issue(tbl_hbm, idxg_v, grp_a, sem_a, off_next)

        _drain(tbl_hbm, grp_b, sem_b)
        _select_write(s_v, grp_b, sel_b, e_out, lane, lane32, base, off_b)
        return carry

    lax.fori_loop(0, NPAIR, step, 0)


def _sc_lin_body(idx_hbm, lin_hbm, lin_out, idx_v, lin_v, sem):
    wid = lax.axis_index("s") * NC + lax.axis_index("c")
    base = wid * PER_W
    pltpu.sync_copy(idx_hbm.at[pl.ds(base, PER_W)], idx_v)

    def step(c, carry):
        off = c * LCHUNK
        idx_c = idx_v.at[pl.ds(off, LCHUNK)]
        pltpu.async_copy(lin_hbm.at[idx_c], lin_v, sem).wait()
        pltpu.sync_copy(lin_v, lin_out.at[pl.ds(base + off, LCHUNK)])
        return carry

    lax.fori_loop(0, NLCHUNK, step, 0)


def _tc_fm_body(e_ref, lin_ref, ra_ref, rb_ref, m_ref, bias_ref, out_ref):
    r = 0.5 * (ra_ref[...] + rb_ref[...])
    ii = lax.broadcasted_iota(jnp.int32, (F, F), 0)
    jj = lax.broadcasted_iota(jnp.int32, (F, F), 1)
    rs = jnp.where(ii == jj, 0.0, r)
    e = e_ref[...]
    t = jnp.dot(rs, e, preferred_element_type=jnp.float32)
    u = jnp.sum(t * e, axis=0, keepdims=True)
    inter = jnp.dot(u, m_ref[...], preferred_element_type=jnp.float32)
    linear = jnp.sum(lin_ref[...], axis=0, keepdims=True)
    out_ref[...] = (inter + linear + bias_ref[0, 0]).reshape(1, 1, 128)


def kernel(token_ids, emb_tables, lin_tables, r_raw, bias):
    tok = token_ids.astype(jnp.int32)
    tokT = tok.T
    idx1d = ((jnp.arange(F, dtype=jnp.int32) * V)[:, None] + tokT).reshape(ROWS)
    idx_g = idx1d >> 3
    idx_s = idx1d & 7

    tbl3 = emb_tables.reshape(F * V // 8, 8, D)
    lin_flat = lin_tables.reshape(F * V)

    mesh = plsc.VectorSubcoreMesh(core_axis_name="c", subcore_axis_name="s")
    gather_e = pl.kernel(
        _sc_emb_body,
        out_type=jax.ShapeDtypeStruct((ROWS * D,), jnp.float32),
        mesh=mesh,
        compiler_params=pltpu.CompilerParams(use_tc_tiling_on_sc=True,
                                             needs_layout_passes=False),
        scratch_types=[
            pltpu.VMEM((PER_W,), jnp.int32),
            pltpu.VMEM((PER_W,), jnp.int32),
            pltpu.VMEM((GCHUNK, 8, D), jnp.float32),
            pltpu.VMEM((GCHUNK, 8, D), jnp.float32),
            pltpu.VMEM((GCHUNK * D,), jnp.float32),
            pltpu.VMEM((GCHUNK * D,), jnp.float32),
            pltpu.SemaphoreType.DMA,
            pltpu.SemaphoreType.DMA,
        ],
    )
    e_rows = gather_e(idx_g, idx_s, tbl3)

    gather_l = pl.kernel(
        _sc_lin_body,
        out_type=jax.ShapeDtypeStruct((ROWS,), jnp.float32),
        mesh=mesh,
        compiler_params=pltpu.CompilerParams(use_tc_tiling_on_sc=False),
        scratch_types=[
            pltpu.VMEM((PER_W,), jnp.int32),
            pltpu.VMEM((LCHUNK,), jnp.float32),
            pltpu.SemaphoreType.DMA,
        ],
    )
    lin_rows = gather_l(idx1d, lin_flat)

    e_mat = e_rows.reshape(F, B * D)
    lin_mat = lin_rows.reshape(F, B)

    msel = ((jnp.arange(128 * D, dtype=jnp.int32) // D)[:, None]
            == jnp.arange(128, dtype=jnp.int32)[None, :]).astype(jnp.float32)

    out3 = pl.pallas_call(
        _tc_fm_body,
        grid=(B // 128,),
        in_specs=[
            pl.BlockSpec((F, 128 * D), lambda i: (0, i)),
            pl.BlockSpec((F, 128), lambda i: (0, i)),
            pl.BlockSpec((F, F), lambda i: (0, 0)),
            pl.BlockSpec((F, F), lambda i: (0, 0)),
            pl.BlockSpec((128 * D, 128), lambda i: (0, 0)),
            pl.BlockSpec((1, 1), lambda i: (0, 0)),
        ],
        out_specs=pl.BlockSpec((1, 1, 128), lambda i: (i, 0, 0)),
        out_shape=jax.ShapeDtypeStruct((B // 128, 1, 128), jnp.float32),
    )(e_mat, lin_mat, r_raw, r_raw.T, msel, bias.reshape(1, 1))

    return out3.reshape(B)
